# Initial kernel scaffold; baseline (speedup 1.0000x reference)
import jax
import jax.numpy as jnp
from jax.experimental import pallas as pl


def kernel(input, weight):
    raise NotImplementedError("write your pallas kernel here")


# speedup vs baseline: 2.9640x; 2.9640x over previous
import functools

import jax
import jax.numpy as jnp
from jax import lax
from jax.experimental import pallas as pl
from jax.experimental.pallas import tpu as pltpu
from jax.experimental.pallas import tpu_sc as plsc

N_V = 100004
N_D = 128
BATCH = 4096
HIST = 50

_B = BATCH * HIST
_NC = 2
_NS = 16
_NW = _NC * _NS
_B_PER_W = _B // _NW
_CH = 128
_NCHUNK = _B_PER_W // _CH


@functools.partial(
    pl.kernel,
    out_type=jax.ShapeDtypeStruct((_B, N_D), jnp.float32),
    mesh=plsc.VectorSubcoreMesh(core_axis_name="c", subcore_axis_name="s"),
    scratch_types=[
        pltpu.VMEM((_B_PER_W,), jnp.int32),
        pltpu.VMEM((_CH, N_D), jnp.float32),
        pltpu.SemaphoreType.DMA,
    ],
)
def _gather_kernel(table_hbm, idx_hbm, out_hbm, idx_v, rows_v, sem):
    wid = lax.axis_index("s") * _NC + lax.axis_index("c")
    base = wid * _B_PER_W
    pltpu.sync_copy(idx_hbm.at[pl.ds(base, _B_PER_W)], idx_v)

    def chunk_body(i, carry):
        off = pl.multiple_of(i * _CH, _CH)
        pltpu.async_copy(
            table_hbm.at[idx_v.at[pl.ds(off, _CH)]], rows_v, sem
        ).wait()
        pltpu.sync_copy(rows_v, out_hbm.at[pl.ds(base + off, _CH)])
        return carry

    lax.fori_loop(0, _NCHUNK, chunk_body, 0)


def kernel(input, weight):
    idx = input.reshape(-1).astype(jnp.int32)
    out = _gather_kernel(weight, idx)
    return out.reshape(BATCH, HIST, N_D)


# speedup vs baseline: 3.3426x; 1.1278x over previous
import functools

import jax
import jax.numpy as jnp
from jax import lax
from jax.experimental import pallas as pl
from jax.experimental.pallas import tpu as pltpu
from jax.experimental.pallas import tpu_sc as plsc

N_V = 100004
N_D = 128
BATCH = 4096
HIST = 50

_B = BATCH * HIST
_NC = 2
_NS = 16
_NW = _NC * _NS
_B_PER_W = _B // _NW
_CH = 128
_NCHUNK = _B_PER_W // _CH
_NBUF = 5
_K = 3
_NOUTER = _NCHUNK // _NBUF


@functools.partial(
    pl.kernel,
    out_type=jax.ShapeDtypeStruct((_B, N_D), jnp.float32),
    mesh=plsc.VectorSubcoreMesh(core_axis_name="c", subcore_axis_name="s"),
    scratch_types=[
        pltpu.VMEM((_B_PER_W,), jnp.int32),
        pltpu.VMEM((_NBUF, _CH, N_D), jnp.float32),
    ]
    + [pltpu.SemaphoreType.DMA] * (2 * _NBUF),
)
def _gather_kernel(table_hbm, idx_hbm, out_hbm, idx_v, rows_v, *sems):
    gsem = sems[:_NBUF]
    ssem = sems[_NBUF:]
    wid = lax.axis_index("s") * _NC + lax.axis_index("c")
    base = wid * _B_PER_W
    pltpu.sync_copy(idx_hbm.at[pl.ds(base, _B_PER_W)], idx_v)

    def g_copy(i, b):
        off = pl.multiple_of(i * _CH, _CH)
        return pltpu.make_async_copy(
            table_hbm.at[idx_v.at[pl.ds(off, _CH)]], rows_v.at[b], gsem[b]
        )

    def s_copy(i, b):
        off = pl.multiple_of(base + i * _CH, _CH)
        return pltpu.make_async_copy(
            rows_v.at[b], out_hbm.at[pl.ds(off, _CH)], ssem[b]
        )

    for j in range(_K):
        g_copy(j, j).start()

    for b in range(_NBUF):
        i = b
        if i - (_NBUF - _K) >= 0:
            s_copy(i - (_NBUF - _K), (i + _K) % _NBUF).wait()
        g_copy(i + _K, (i + _K) % _NBUF).start()
        g_copy(i, b).wait()
        s_copy(i, b).start()

    def outer_body(o, carry):
        for b in range(_NBUF):
            i = o * _NBUF + b
            s_copy(i - (_NBUF - _K), (b + _K) % _NBUF).wait()
            g_copy(i + _K, (b + _K) % _NBUF).start()
            g_copy(i, b).wait()
            s_copy(i, b).start()
        return carry

    lax.fori_loop(1, _NOUTER - 1, outer_body, 0)

    for b in range(_NBUF):
        i = _NCHUNK - _NBUF + b
        s_copy(i - (_NBUF - _K), (i + _K) % _NBUF).wait()
        if i + _K < _NCHUNK:
            g_copy(i + _K, (i + _K) % _NBUF).start()
        g_copy(i, b).wait()
        s_copy(i, b).start()

    for j in range(_NBUF - _K):
        i = _NCHUNK - (_NBUF - _K) + j
        s_copy(i, i % _NBUF).wait()


def kernel(input, weight):
    idx = input.reshape(-1).astype(jnp.int32)
    out = _gather_kernel(weight, idx)
    return out.reshape(BATCH, HIST, N_D)


# speedup vs baseline: 3.3444x; 1.0005x over previous
import functools

import jax
import jax.numpy as jnp
from jax import lax
from jax.experimental import pallas as pl
from jax.experimental.pallas import tpu as pltpu
from jax.experimental.pallas import tpu_sc as plsc

N_V = 100004
N_D = 128
BATCH = 4096
HIST = 50

_B = BATCH * HIST
_NC = 2
_NS = 16
_NW = _NC * _NS
_B_PER_W = _B // _NW
_CH = 128
_NCHUNK = _B_PER_W // _CH
_NBUF = 5
_K = 4
_NOUTER = _NCHUNK // _NBUF


@functools.partial(
    pl.kernel,
    out_type=jax.ShapeDtypeStruct((_B, N_D), jnp.float32),
    mesh=plsc.VectorSubcoreMesh(core_axis_name="c", subcore_axis_name="s"),
    scratch_types=[
        pltpu.VMEM((_B_PER_W,), jnp.int32),
        pltpu.VMEM((_NBUF, _CH, N_D), jnp.float32),
    ]
    + [pltpu.SemaphoreType.DMA] * (2 * _NBUF),
)
def _gather_kernel(table_hbm, idx_hbm, out_hbm, idx_v, rows_v, *sems):
    gsem = sems[:_NBUF]
    ssem = sems[_NBUF:]
    wid = lax.axis_index("s") * _NC + lax.axis_index("c")
    base = wid * _B_PER_W
    pltpu.sync_copy(idx_hbm.at[pl.ds(base, _B_PER_W)], idx_v)

    def g_copy(i, b):
        off = pl.multiple_of(i * _CH, _CH)
        return pltpu.make_async_copy(
            table_hbm.at[idx_v.at[pl.ds(off, _CH)]], rows_v.at[b], gsem[b]
        )

    def s_copy(i, b):
        off = pl.multiple_of(base + i * _CH, _CH)
        return pltpu.make_async_copy(
            rows_v.at[b], out_hbm.at[pl.ds(off, _CH)], ssem[b]
        )

    for j in range(_K):
        g_copy(j, j).start()

    for b in range(_NBUF):
        i = b
        if i - (_NBUF - _K) >= 0:
            s_copy(i - (_NBUF - _K), (i + _K) % _NBUF).wait()
        g_copy(i + _K, (i + _K) % _NBUF).start()
        g_copy(i, b).wait()
        s_copy(i, b).start()

    def outer_body(o, carry):
        for b in range(_NBUF):
            i = o * _NBUF + b
            s_copy(i - (_NBUF - _K), (b + _K) % _NBUF).wait()
            g_copy(i + _K, (b + _K) % _NBUF).start()
            g_copy(i, b).wait()
            s_copy(i, b).start()
        return carry

    lax.fori_loop(1, _NOUTER - 1, outer_body, 0)

    for b in range(_NBUF):
        i = _NCHUNK - _NBUF + b
        s_copy(i - (_NBUF - _K), (i + _K) % _NBUF).wait()
        if i + _K < _NCHUNK:
            g_copy(i + _K, (i + _K) % _NBUF).start()
        g_copy(i, b).wait()
        s_copy(i, b).start()

    for j in range(_NBUF - _K):
        i = _NCHUNK - (_NBUF - _K) + j
        s_copy(i, i % _NBUF).wait()


def kernel(input, weight):
    idx = input.reshape(-1).astype(jnp.int32)
    out = _gather_kernel(weight, idx)
    return out.reshape(BATCH, HIST, N_D)


# speedup vs baseline: 10.7577x; 3.2166x over previous
import functools

import jax
import jax.numpy as jnp
from jax import lax
from jax.experimental import pallas as pl
from jax.experimental.pallas import tpu as pltpu
from jax.experimental.pallas import tpu_sc as plsc

N_V = 100004
N_D = 128
BATCH = 4096
HIST = 50

_B = BATCH * HIST
_NC = 2
_NS = 16
_NW = _NC * _NS
_CB = BATCH // _NW
_NCHUNK = HIST
_NBUF = 5
_K = 3
_NOUTER = _NCHUNK // _NBUF


@functools.partial(
    pl.kernel,
    out_type=jax.ShapeDtypeStruct((_B, N_D), jnp.float32),
    mesh=plsc.VectorSubcoreMesh(core_axis_name="c", subcore_axis_name="s"),
    scratch_types=[
        pltpu.VMEM((_NCHUNK, _CB), jnp.int32),
        pltpu.VMEM((_NBUF, _CB, N_D), jnp.float32),
    ]
    + [pltpu.SemaphoreType.DMA] * (2 * _NBUF),
)
def _gather_kernel(table_hbm, idx_hbm, out_hbm, idx_v, rows_v, *sems):
    gsem = sems[:_NBUF]
    ssem = sems[_NBUF:]
    wid = lax.axis_index("s") * _NC + lax.axis_index("c")
    col = wid * _CB
    pltpu.sync_copy(idx_hbm.at[:, pl.ds(col, _CB)], idx_v)

    def g_copy(h, b):
        return pltpu.make_async_copy(
            table_hbm.at[idx_v.at[h]], rows_v.at[b], gsem[b]
        )

    def s_copy(h, b):
        off = pl.multiple_of(h * BATCH + col, _CB)
        return pltpu.make_async_copy(
            rows_v.at[b], out_hbm.at[pl.ds(off, _CB)], ssem[b]
        )

    for j in range(_K):
        g_copy(j, j).start()

    for b in range(_NBUF):
        i = b
        if i - (_NBUF - _K) >= 0:
            s_copy(i - (_NBUF - _K), (i + _K) % _NBUF).wait()
        g_copy(i + _K, (i + _K) % _NBUF).start()
        g_copy(i, b).wait()
        s_copy(i, b).start()

    def outer_body(o, carry):
        for b in range(_NBUF):
            i = o * _NBUF + b
            s_copy(i - (_NBUF - _K), (b + _K) % _NBUF).wait()
            g_copy(i + _K, (b + _K) % _NBUF).start()
            g_copy(i, b).wait()
            s_copy(i, b).start()
        return carry

    lax.fori_loop(1, _NOUTER - 1, outer_body, 0)

    for b in range(_NBUF):
        i = _NCHUNK - _NBUF + b
        s_copy(i - (_NBUF - _K), (i + _K) % _NBUF).wait()
        if i + _K < _NCHUNK:
            g_copy(i + _K, (i + _K) % _NBUF).start()
        g_copy(i, b).wait()
        s_copy(i, b).start()

    for j in range(_NBUF - _K):
        i = _NCHUNK - (_NBUF - _K) + j
        s_copy(i, i % _NBUF).wait()


def kernel(input, weight):
    idx_t = input.astype(jnp.int32).T
    out = _gather_kernel(weight, idx_t)
    return out.reshape(HIST, BATCH, N_D).transpose(1, 0, 2)
